# Initial kernel scaffold; baseline (speedup 1.0000x reference)
import jax
import jax.numpy as jnp
from jax.experimental import pallas as pl


def kernel(x, adj_mat, U):
    raise NotImplementedError("write your pallas kernel here")


# speedup vs baseline: 1.4752x; 1.4752x over previous
import jax
import jax.numpy as jnp
from jax.experimental import pallas as pl
from jax.experimental.pallas import tpu as pltpu

_N = 8192
_D = 128
_BI = 512
_BK = 2048
_MH = 144


def _body(adj_ref, xh_ref, xl_ref, u_ref, out_ref, acc_ref):
    k = pl.program_id(1)
    nk = pl.num_programs(1)

    @pl.when(k == 0)
    def _init():
        acc_ref[...] = jnp.zeros_like(acc_ref)

    a = adj_ref[...].astype(jnp.bfloat16)
    xh = xh_ref[:, pl.ds(k * _BK, _BK)]
    xl = xl_ref[:, pl.ds(k * _BK, _BK)]
    dims = (((1,), (0,)), ((), ()))
    hi = jax.lax.dot_general(xh, a, dims, preferred_element_type=jnp.float32)
    lo = jax.lax.dot_general(xl, a, dims, preferred_element_type=jnp.float32)
    acc_ref[...] += hi + lo

    @pl.when(k == nk - 1)
    def _epilogue():
        agg = acc_ref[0:_D, :]
        deg = acc_ref[_D:_D + 1, :]
        y = jnp.dot(u_ref[...], agg, preferred_element_type=jnp.float32)
        y = jnp.maximum(y / deg, 0.0)
        out_ref[...] = y.T


def kernel(x, adj_mat, U):
    xt = x.T
    xh = xt.astype(jnp.bfloat16)
    xl = (xt - xh.astype(jnp.float32)).astype(jnp.bfloat16)
    ones_row = jnp.ones((1, _N), jnp.bfloat16)
    zpad = jnp.zeros((_MH - _D - 1, _N), jnp.bfloat16)
    xh_aug = jnp.concatenate([xh, ones_row, zpad], axis=0)
    xl_aug = jnp.concatenate([xl, jnp.zeros((_MH - _D, _N), jnp.bfloat16)],
                             axis=0)

    grid = (_N // _BI, _N // _BK)
    return pl.pallas_call(
        _body,
        grid=grid,
        in_specs=[
            pl.BlockSpec((_BK, _BI), lambda i, k: (k, i)),
            pl.BlockSpec((_MH, _N), lambda i, k: (0, 0)),
            pl.BlockSpec((_MH, _N), lambda i, k: (0, 0)),
            pl.BlockSpec((_D, _D), lambda i, k: (0, 0)),
        ],
        out_specs=pl.BlockSpec((_BI, _D), lambda i, k: (i, 0)),
        out_shape=jax.ShapeDtypeStruct((_N, _D), jnp.float32),
        scratch_shapes=[pltpu.VMEM((_MH, _BI), jnp.float32)],
        compiler_params=pltpu.CompilerParams(
            dimension_semantics=("parallel", "arbitrary")),
    )(adj_mat, xh_aug, xl_aug, U)


# speedup vs baseline: 1.7651x; 1.1966x over previous
import jax
import jax.numpy as jnp
from jax.experimental import pallas as pl
from jax.experimental.pallas import tpu as pltpu

_N = 8192
_D = 128
_BI = 1024
_BK = 2048
_MH = 144


def _body(adj_ref, xh_ref, xl_ref, u_ref, out_ref, acc_ref):
    k = pl.program_id(1)
    nk = pl.num_programs(1)

    @pl.when(k == 0)
    def _init():
        acc_ref[...] = jnp.zeros_like(acc_ref)

    a = adj_ref[...].astype(jnp.bfloat16)
    xh = xh_ref[:, pl.ds(k * _BK, _BK)]
    xl = xl_ref[:, pl.ds(k * _BK, _BK)]
    dims = (((1,), (0,)), ((), ()))
    hi = jax.lax.dot_general(xh, a, dims, preferred_element_type=jnp.float32)
    lo = jax.lax.dot_general(xl, a, dims, preferred_element_type=jnp.float32)
    acc_ref[...] += hi + lo

    @pl.when(k == nk - 1)
    def _epilogue():
        agg = acc_ref[0:_D, :]
        deg = acc_ref[_D:_D + 1, :]
        y = jnp.dot(u_ref[...], agg, preferred_element_type=jnp.float32)
        y = jnp.maximum(y / deg, 0.0)
        out_ref[...] = y.T


def kernel(x, adj_mat, U):
    xt = x.T
    xh = xt.astype(jnp.bfloat16)
    xl = (xt - xh.astype(jnp.float32)).astype(jnp.bfloat16)
    ones_row = jnp.ones((1, _N), jnp.bfloat16)
    zpad = jnp.zeros((_MH - _D - 1, _N), jnp.bfloat16)
    xh_aug = jnp.concatenate([xh, ones_row, zpad], axis=0)
    xl_aug = jnp.concatenate([xl, jnp.zeros((_MH - _D, _N), jnp.bfloat16)],
                             axis=0)

    grid = (_N // _BI, _N // _BK)
    return pl.pallas_call(
        _body,
        grid=grid,
        in_specs=[
            pl.BlockSpec((_BK, _BI), lambda i, k: (k, i)),
            pl.BlockSpec((_MH, _N), lambda i, k: (0, 0)),
            pl.BlockSpec((_MH, _N), lambda i, k: (0, 0)),
            pl.BlockSpec((_D, _D), lambda i, k: (0, 0)),
        ],
        out_specs=pl.BlockSpec((_BI, _D), lambda i, k: (i, 0)),
        out_shape=jax.ShapeDtypeStruct((_N, _D), jnp.float32),
        scratch_shapes=[pltpu.VMEM((_MH, _BI), jnp.float32)],
        compiler_params=pltpu.CompilerParams(
            dimension_semantics=("parallel", "arbitrary")),
    )(adj_mat, xh_aug, xl_aug, U)


# speedup vs baseline: 1.7845x; 1.0110x over previous
import jax
import jax.numpy as jnp
from jax.experimental import pallas as pl
from jax.experimental.pallas import tpu as pltpu

_N = 8192
_D = 128
_BI = 2048
_BK = 1024
_MH = 144


def _body(adj_ref, xh_ref, xl_ref, u_ref, out_ref, acc_ref):
    k = pl.program_id(1)
    nk = pl.num_programs(1)

    @pl.when(k == 0)
    def _init():
        acc_ref[...] = jnp.zeros_like(acc_ref)

    a = adj_ref[...].astype(jnp.bfloat16)
    xh = xh_ref[:, pl.ds(k * _BK, _BK)]
    xl = xl_ref[:, pl.ds(k * _BK, _BK)]
    dims = (((1,), (0,)), ((), ()))
    hi = jax.lax.dot_general(xh, a, dims, preferred_element_type=jnp.float32)
    lo = jax.lax.dot_general(xl, a, dims, preferred_element_type=jnp.float32)
    acc_ref[...] += hi + lo

    @pl.when(k == nk - 1)
    def _epilogue():
        agg = acc_ref[0:_D, :]
        deg = acc_ref[_D:_D + 1, :]
        y = jnp.dot(u_ref[...], agg, preferred_element_type=jnp.float32)
        y = jnp.maximum(y / deg, 0.0)
        out_ref[...] = y.T


def kernel(x, adj_mat, U):
    xt = x.T
    xh = xt.astype(jnp.bfloat16)
    xl = (xt - xh.astype(jnp.float32)).astype(jnp.bfloat16)
    ones_row = jnp.ones((1, _N), jnp.bfloat16)
    zpad = jnp.zeros((_MH - _D - 1, _N), jnp.bfloat16)
    xh_aug = jnp.concatenate([xh, ones_row, zpad], axis=0)
    xl_aug = jnp.concatenate([xl, jnp.zeros((_MH - _D, _N), jnp.bfloat16)],
                             axis=0)

    grid = (_N // _BI, _N // _BK)
    return pl.pallas_call(
        _body,
        grid=grid,
        in_specs=[
            pl.BlockSpec((_BK, _BI), lambda i, k: (k, i)),
            pl.BlockSpec((_MH, _N), lambda i, k: (0, 0)),
            pl.BlockSpec((_MH, _N), lambda i, k: (0, 0)),
            pl.BlockSpec((_D, _D), lambda i, k: (0, 0)),
        ],
        out_specs=pl.BlockSpec((_BI, _D), lambda i, k: (i, 0)),
        out_shape=jax.ShapeDtypeStruct((_N, _D), jnp.float32),
        scratch_shapes=[pltpu.VMEM((_MH, _BI), jnp.float32)],
        compiler_params=pltpu.CompilerParams(
            dimension_semantics=("parallel", "arbitrary")),
    )(adj_mat, xh_aug, xl_aug, U)


# speedup vs baseline: 1.7922x; 1.0043x over previous
import jax
import jax.numpy as jnp
from jax.experimental import pallas as pl
from jax.experimental.pallas import tpu as pltpu

_N = 8192
_D = 128
_BK = 256
_MH = 144


def _body(adj_ref, xh_ref, xl_ref, u_ref, out_ref, acc_ref):
    k = pl.program_id(0)
    nk = pl.num_programs(0)

    @pl.when(k == 0)
    def _init():
        acc_ref[...] = jnp.zeros_like(acc_ref)

    a = adj_ref[...].astype(jnp.bfloat16)
    xh = xh_ref[:, pl.ds(k * _BK, _BK)]
    xl = xl_ref[:, pl.ds(k * _BK, _BK)]
    dims = (((1,), (0,)), ((), ()))
    hi = jax.lax.dot_general(xh, a, dims, preferred_element_type=jnp.float32)
    lo = jax.lax.dot_general(xl, a, dims, preferred_element_type=jnp.float32)
    acc_ref[...] += hi + lo

    @pl.when(k == nk - 1)
    def _epilogue():
        agg = acc_ref[0:_D, :]
        deg = acc_ref[_D:_D + 1, :]
        y = jnp.dot(u_ref[...], agg, preferred_element_type=jnp.float32)
        y = jnp.maximum(y / deg, 0.0)
        out_ref[...] = y.T


def kernel(x, adj_mat, U):
    xt = x.T
    xh = xt.astype(jnp.bfloat16)
    xl = (xt - xh.astype(jnp.float32)).astype(jnp.bfloat16)
    ones_row = jnp.ones((1, _N), jnp.bfloat16)
    zpad = jnp.zeros((_MH - _D - 1, _N), jnp.bfloat16)
    xh_aug = jnp.concatenate([xh, ones_row, zpad], axis=0)
    xl_aug = jnp.concatenate([xl, jnp.zeros((_MH - _D, _N), jnp.bfloat16)],
                             axis=0)

    return pl.pallas_call(
        _body,
        grid=(_N // _BK,),
        in_specs=[
            pl.BlockSpec((_BK, _N), lambda k: (k, 0)),
            pl.BlockSpec((_MH, _N), lambda k: (0, 0)),
            pl.BlockSpec((_MH, _N), lambda k: (0, 0)),
            pl.BlockSpec((_D, _D), lambda k: (0, 0)),
        ],
        out_specs=pl.BlockSpec((_N, _D), lambda k: (0, 0)),
        out_shape=jax.ShapeDtypeStruct((_N, _D), jnp.float32),
        scratch_shapes=[pltpu.VMEM((_MH, _N), jnp.float32)],
        compiler_params=pltpu.CompilerParams(
            dimension_semantics=("arbitrary",)),
    )(adj_mat, xh_aug, xl_aug, U)


# speedup vs baseline: 1.8024x; 1.0057x over previous
import jax
import jax.numpy as jnp
from jax.experimental import pallas as pl
from jax.experimental.pallas import tpu as pltpu

_N = 8192
_D = 128
_BK = 512
_MH = 144


def _body(adj_ref, xh_ref, xl_ref, u_ref, out_ref, acc_ref):
    k = pl.program_id(0)
    nk = pl.num_programs(0)

    @pl.when(k == 0)
    def _init():
        acc_ref[...] = jnp.zeros_like(acc_ref)

    a = adj_ref[...].astype(jnp.bfloat16)
    xh = xh_ref[:, pl.ds(k * _BK, _BK)]
    xl = xl_ref[:, pl.ds(k * _BK, _BK)]
    dims = (((1,), (0,)), ((), ()))
    hi = jax.lax.dot_general(xh, a, dims, preferred_element_type=jnp.float32)
    lo = jax.lax.dot_general(xl, a, dims, preferred_element_type=jnp.float32)
    acc_ref[...] += hi + lo

    @pl.when(k == nk - 1)
    def _epilogue():
        agg = acc_ref[0:_D, :]
        deg = acc_ref[_D:_D + 1, :]
        y = jnp.dot(u_ref[...], agg, preferred_element_type=jnp.float32)
        y = jnp.maximum(y / deg, 0.0)
        out_ref[...] = y.T


def kernel(x, adj_mat, U):
    xt = x.T
    xh = xt.astype(jnp.bfloat16)
    xl = (xt - xh.astype(jnp.float32)).astype(jnp.bfloat16)
    ones_row = jnp.ones((1, _N), jnp.bfloat16)
    zpad = jnp.zeros((_MH - _D - 1, _N), jnp.bfloat16)
    xh_aug = jnp.concatenate([xh, ones_row, zpad], axis=0)
    xl_aug = jnp.concatenate([xl, jnp.zeros((_MH - _D, _N), jnp.bfloat16)],
                             axis=0)

    return pl.pallas_call(
        _body,
        grid=(_N // _BK,),
        in_specs=[
            pl.BlockSpec((_BK, _N), lambda k: (k, 0)),
            pl.BlockSpec((_MH, _N), lambda k: (0, 0)),
            pl.BlockSpec((_MH, _N), lambda k: (0, 0)),
            pl.BlockSpec((_D, _D), lambda k: (0, 0)),
        ],
        out_specs=pl.BlockSpec((_N, _D), lambda k: (0, 0)),
        out_shape=jax.ShapeDtypeStruct((_N, _D), jnp.float32),
        scratch_shapes=[pltpu.VMEM((_MH, _N), jnp.float32)],
        compiler_params=pltpu.CompilerParams(
            dimension_semantics=("arbitrary",)),
    )(adj_mat, xh_aug, xl_aug, U)


# speedup vs baseline: 1.8592x; 1.0315x over previous
import jax
import jax.numpy as jnp
from jax.experimental import pallas as pl
from jax.experimental.pallas import tpu as pltpu

_N = 8192
_D = 128
_BK = 512
_MH = 136


def _body(adj_ref, xh_ref, u_ref, out_ref, acc_ref):
    k = pl.program_id(0)
    nk = pl.num_programs(0)

    @pl.when(k == 0)
    def _init():
        acc_ref[...] = jnp.zeros_like(acc_ref)

    a = adj_ref[...].astype(jnp.bfloat16)
    xh = xh_ref[:, pl.ds(k * _BK, _BK)]
    dims = (((1,), (0,)), ((), ()))
    hi = jax.lax.dot_general(xh, a, dims, preferred_element_type=jnp.float32)
    acc_ref[...] += hi

    @pl.when(k == nk - 1)
    def _epilogue():
        agg = acc_ref[0:_D, :]
        deg = acc_ref[_D:_D + 1, :]
        y = jnp.dot(u_ref[...], agg, preferred_element_type=jnp.float32)
        y = jnp.maximum(y / deg, 0.0)
        out_ref[...] = y.T


def kernel(x, adj_mat, U):
    xh = x.T.astype(jnp.bfloat16)
    ones_row = jnp.ones((1, _N), jnp.bfloat16)
    zpad = jnp.zeros((_MH - _D - 1, _N), jnp.bfloat16)
    xh_aug = jnp.concatenate([xh, ones_row, zpad], axis=0)

    return pl.pallas_call(
        _body,
        grid=(_N // _BK,),
        in_specs=[
            pl.BlockSpec((_BK, _N), lambda k: (k, 0)),
            pl.BlockSpec((_MH, _N), lambda k: (0, 0)),
            pl.BlockSpec((_D, _D), lambda k: (0, 0)),
        ],
        out_specs=pl.BlockSpec((_N, _D), lambda k: (0, 0)),
        out_shape=jax.ShapeDtypeStruct((_N, _D), jnp.float32),
        scratch_shapes=[pltpu.VMEM((_MH, _N), jnp.float32)],
        compiler_params=pltpu.CompilerParams(
            dimension_semantics=("arbitrary",)),
    )(adj_mat, xh_aug, U)
